# baseline (speedup 1.0000x reference)
import functools

import jax
import jax.numpy as jnp
from jax import lax
from jax.experimental import pallas as pl
from jax.experimental.pallas import tpu as pltpu
from jax.experimental.pallas import tpu_sc as plsc

EN = 512
ED = 64
TOK = 32 * 28 * 28
TBLK = 3136
GRID = TOK // TBLK


def _conv2d(x, w, b, stride, pad):
    y = lax.conv_general_dilated(
        x, w, (stride, stride), [(pad, pad), (pad, pad)],
        dimension_numbers=('NCHW', 'OIHW', 'NCHW'))
    return y + b[None, :, None, None]


def _conv_transpose2d(x, w, b, stride, pad):
    k = w.shape[2]
    wc = jnp.flip(jnp.transpose(w, (1, 0, 2, 3)), axis=(2, 3))
    pp = k - 1 - pad
    n, c, h, wd = x.shape
    xd = jnp.zeros((n, c, (h - 1) * stride + 1, (wd - 1) * stride + 1), x.dtype)
    xd = xd.at[:, :, ::stride, ::stride].set(x)
    y = lax.conv_general_dilated(
        xd, wc, (1, 1), [(pp, pp), (pp, pp)],
        dimension_numbers=('NCHW', 'OIHW', 'NCHW'))
    return y + b[None, :, None, None]


def _batchnorm(x, gamma, beta, eps=1e-5):
    mean = jnp.mean(x, axis=(0, 2, 3), keepdims=True)
    var = jnp.var(x, axis=(0, 2, 3), keepdims=True)
    xh = (x - mean) * lax.rsqrt(var + eps)
    return xh * gamma[None, :, None, None] + beta[None, :, None, None]


def _vq_body(zf_ref, cbt_ref, zsq_ref, cbsq_ref, idx_ref, loss_ref):
    zf = zf_ref[...]
    mm = jnp.dot(zf, cbt_ref[...], preferred_element_type=jnp.float32)
    d = (zsq_ref[...] + cbsq_ref[...]) - 2.0 * mm
    dmin = jnp.min(d, axis=1, keepdims=True)
    lanes = lax.broadcasted_iota(jnp.int32, d.shape, 1)
    idx = jnp.min(jnp.where(d == dmin, lanes, EN), axis=1)
    idx_ref[0, 0, :] = idx

    @pl.when(pl.program_id(0) == 0)
    def _init():
        loss_ref[...] = jnp.zeros((1, 1), jnp.float32)
    loss_ref[...] += jnp.sum(dmin, keepdims=True)


def _vq_tc(z_flat, cbt, zsq, cbsq):
    return pl.pallas_call(
        _vq_body,
        grid=(GRID,),
        in_specs=[
            pl.BlockSpec((TBLK, ED), lambda i: (i, 0)),
            pl.BlockSpec((ED, EN), lambda i: (0, 0)),
            pl.BlockSpec((TBLK, 1), lambda i: (i, 0)),
            pl.BlockSpec((1, EN), lambda i: (0, 0)),
        ],
        out_specs=[
            pl.BlockSpec((1, 1, TBLK), lambda i: (i, 0, 0)),
            pl.BlockSpec((1, 1), lambda i: (0, 0)),
        ],
        out_shape=[
            jax.ShapeDtypeStruct((GRID, 1, TBLK), jnp.int32),
            jax.ShapeDtypeStruct((1, 1), jnp.float32),
        ],
    )(z_flat, cbt, zsq, cbsq)


@functools.lru_cache(maxsize=1)
def _sc_gather_fn():
    info = plsc.get_sparse_core_info()
    nw = info.num_cores * info.num_subcores
    bpw = TOK // nw
    mesh = plsc.VectorSubcoreMesh(core_axis_name="c", subcore_axis_name="s")
    nc = info.num_cores

    @functools.partial(
        pl.kernel, mesh=mesh,
        compiler_params=pltpu.CompilerParams(use_tc_tiling_on_sc=False),
        out_type=jax.ShapeDtypeStruct((TOK, ED), jnp.float32),
        scratch_types=[
            pltpu.VMEM((bpw,), jnp.int32),
            pltpu.VMEM((bpw, ED), jnp.float32),
            pltpu.SemaphoreType.DMA,
        ],
    )
    def gather(cb_hbm, idx_hbm, out_hbm, idx_v, rows_v, sem):
        wid = lax.axis_index("s") * nc + lax.axis_index("c")
        base = wid * bpw
        pltpu.sync_copy(idx_hbm.at[pl.ds(base, bpw)], idx_v)
        pltpu.async_copy(cb_hbm.at[idx_v], rows_v, sem).wait()
        pltpu.sync_copy(rows_v, out_hbm.at[pl.ds(base, bpw)])

    return gather


def kernel(x, codebook, ew1, eb1, eg1, ebt1, ew2, eb2, eg2, ebt2, ew3, eb3,
           eg3, ebt3, dw1, db1, dg1, dbt1, dw2, db2, dg2, dbt2, dw3, db3,
           dg3, dbt3):
    z = jax.nn.relu(_batchnorm(_conv2d(x, ew1, eb1, 2, 1), eg1, ebt1))
    z = jax.nn.relu(_batchnorm(_conv2d(z, ew2, eb2, 2, 1), eg2, ebt2))
    z_e = jax.nn.relu(_batchnorm(_conv2d(z, ew3, eb3, 2, 1), eg3, ebt3))

    z_perm = jnp.transpose(z_e, (0, 2, 3, 1))
    z_flat = z_perm.reshape(-1, ED)
    zsq = jnp.sum(z_flat ** 2, axis=1, keepdims=True)
    cbsq = jnp.sum(codebook ** 2, axis=1)[None, :]
    cbt = codebook.T

    idx3, loss_sum = _vq_tc(z_flat, cbt, zsq, cbsq)
    idx = idx3.reshape(TOK)
    z_quant_flat = _sc_gather_fn()(codebook, idx)

    loss_vq = 2.0 * loss_sum[0, 0] / (TOK * ED)
    z_q = jnp.transpose(z_quant_flat.reshape(z_perm.shape), (0, 3, 1, 2))

    y = jax.nn.relu(_batchnorm(_conv_transpose2d(z_q, dw1, db1, 2, 1), dg1, dbt1))
    y = jax.nn.relu(_batchnorm(_conv_transpose2d(y, dw2, db2, 2, 1), dg2, dbt2))
    z_d = jnp.tanh(_batchnorm(_conv_transpose2d(y, dw3, db3, 2, 1), dg3, dbt3))
    return (z_d, idx[:, None], loss_vq)


# speedup vs baseline: 2.0726x; 2.0726x over previous
import functools

import jax
import jax.numpy as jnp
from jax import lax
from jax.experimental import pallas as pl
from jax.experimental.pallas import tpu as pltpu
from jax.experimental.pallas import tpu_sc as plsc

EN = 512
ED = 64
TOK = 32 * 28 * 28
TBLK = 3136
GRID = TOK // TBLK


def _conv2d(x, w, b, stride, pad):
    y = lax.conv_general_dilated(
        x, w, (stride, stride), [(pad, pad), (pad, pad)],
        dimension_numbers=('NCHW', 'OIHW', 'NCHW'))
    return y + b[None, :, None, None]


def _conv_transpose2d_nhwc(x, w, b, stride, pad):
    k = w.shape[2]
    wc = jnp.flip(jnp.transpose(w, (2, 3, 0, 1)), axis=(0, 1))
    pp = k - 1 - pad
    y = lax.conv_general_dilated(
        x, wc, (1, 1), [(pp, pp), (pp, pp)],
        lhs_dilation=(stride, stride),
        dimension_numbers=('NHWC', 'HWIO', 'NHWC'))
    return y + b[None, None, None, :]


def _batchnorm(x, gamma, beta, eps=1e-5):
    mean = jnp.mean(x, axis=(0, 2, 3), keepdims=True)
    var = jnp.var(x, axis=(0, 2, 3), keepdims=True)
    xh = (x - mean) * lax.rsqrt(var + eps)
    return xh * gamma[None, :, None, None] + beta[None, :, None, None]


def _batchnorm_nhwc(x, gamma, beta, eps=1e-5):
    mean = jnp.mean(x, axis=(0, 1, 2), keepdims=True)
    var = jnp.var(x, axis=(0, 1, 2), keepdims=True)
    xh = (x - mean) * lax.rsqrt(var + eps)
    return xh * gamma[None, None, None, :] + beta[None, None, None, :]


def _vq_body(zf_ref, cbt_ref, zsq_ref, cbsq_ref, idx_ref, loss_ref):
    zf = zf_ref[...]
    mm = jnp.dot(zf, cbt_ref[...], preferred_element_type=jnp.float32)
    d = (zsq_ref[...] + cbsq_ref[...]) - 2.0 * mm
    dmin = jnp.min(d, axis=1, keepdims=True)
    lanes = lax.broadcasted_iota(jnp.int32, d.shape, 1)
    idx = jnp.min(jnp.where(d == dmin, lanes, EN), axis=1)
    idx_ref[0, 0, :] = idx

    @pl.when(pl.program_id(0) == 0)
    def _init():
        loss_ref[...] = jnp.zeros((1, 1), jnp.float32)
    loss_ref[...] += jnp.sum(dmin, keepdims=True)


def _vq_tc(z_flat, cbt, zsq, cbsq):
    return pl.pallas_call(
        _vq_body,
        grid=(GRID,),
        in_specs=[
            pl.BlockSpec((TBLK, ED), lambda i: (i, 0)),
            pl.BlockSpec((ED, EN), lambda i: (0, 0)),
            pl.BlockSpec((TBLK, 1), lambda i: (i, 0)),
            pl.BlockSpec((1, EN), lambda i: (0, 0)),
        ],
        out_specs=[
            pl.BlockSpec((1, 1, TBLK), lambda i: (i, 0, 0)),
            pl.BlockSpec((1, 1), lambda i: (0, 0)),
        ],
        out_shape=[
            jax.ShapeDtypeStruct((GRID, 1, TBLK), jnp.int32),
            jax.ShapeDtypeStruct((1, 1), jnp.float32),
        ],
    )(z_flat, cbt, zsq, cbsq)


@functools.lru_cache(maxsize=1)
def _sc_gather_fn():
    info = plsc.get_sparse_core_info()
    nw = info.num_cores * info.num_subcores
    bpw = TOK // nw
    mesh = plsc.VectorSubcoreMesh(core_axis_name="c", subcore_axis_name="s")
    nc = info.num_cores

    @functools.partial(
        pl.kernel, mesh=mesh,
        compiler_params=pltpu.CompilerParams(use_tc_tiling_on_sc=False),
        out_type=jax.ShapeDtypeStruct((TOK, ED), jnp.float32),
        scratch_types=[
            pltpu.VMEM((bpw,), jnp.int32),
            pltpu.VMEM((bpw, ED), jnp.float32),
            pltpu.SemaphoreType.DMA,
        ],
    )
    def gather(cb_hbm, idx_hbm, out_hbm, idx_v, rows_v, sem):
        wid = lax.axis_index("s") * nc + lax.axis_index("c")
        base = wid * bpw
        pltpu.sync_copy(idx_hbm.at[pl.ds(base, bpw)], idx_v)
        pltpu.async_copy(cb_hbm.at[idx_v], rows_v, sem).wait()
        pltpu.sync_copy(rows_v, out_hbm.at[pl.ds(base, bpw)])

    return gather


def kernel(x, codebook, ew1, eb1, eg1, ebt1, ew2, eb2, eg2, ebt2, ew3, eb3,
           eg3, ebt3, dw1, db1, dg1, dbt1, dw2, db2, dg2, dbt2, dw3, db3,
           dg3, dbt3):
    z = jax.nn.relu(_batchnorm(_conv2d(x, ew1, eb1, 2, 1), eg1, ebt1))
    z = jax.nn.relu(_batchnorm(_conv2d(z, ew2, eb2, 2, 1), eg2, ebt2))
    z_e = jax.nn.relu(_batchnorm(_conv2d(z, ew3, eb3, 2, 1), eg3, ebt3))

    z_perm = jnp.transpose(z_e, (0, 2, 3, 1))
    z_flat = z_perm.reshape(-1, ED)
    zsq = jnp.sum(z_flat ** 2, axis=1, keepdims=True)
    cbsq = jnp.sum(codebook ** 2, axis=1)[None, :]
    cbt = codebook.T

    idx3, loss_sum = _vq_tc(z_flat, cbt, zsq, cbsq)
    idx = idx3.reshape(TOK)
    z_quant_flat = _sc_gather_fn()(codebook, idx)

    loss_vq = 2.0 * loss_sum[0, 0] / (TOK * ED)
    z_q = z_quant_flat.reshape(z_perm.shape)

    y = jax.nn.relu(_batchnorm_nhwc(
        _conv_transpose2d_nhwc(z_q, dw1, db1, 2, 1), dg1, dbt1))
    y = jax.nn.relu(_batchnorm_nhwc(
        _conv_transpose2d_nhwc(y, dw2, db2, 2, 1), dg2, dbt2))
    z_d = jnp.tanh(_batchnorm_nhwc(
        _conv_transpose2d_nhwc(y, dw3, db3, 2, 1), dg3, dbt3))
    z_d = jnp.transpose(z_d, (0, 3, 1, 2))
    return (z_d, idx[:, None], loss_vq)
